# baseline (speedup 1.0000x reference)
import functools

import jax
import jax.numpy as jnp
from jax import lax
from jax.experimental import pallas as pl
from jax.experimental.pallas import tpu as pltpu
from jax.experimental.pallas import tpu_sc as plsc

BATCH = 16384
EMBED_DIM = 64

_info = plsc.get_sparse_core_info()
_NC, _NS, _L = _info.num_cores, _info.num_subcores, _info.num_lanes
_NW = _NC * _NS
_BPW = BATCH // _NW
_CHUNKS = _BPW // _L


def _sc_body(uidx_hbm, midx_hbm, utab_hbm, mtab_hbm, out_hbm,
             uidx_v, midx_v, urows_v, mrows_v, out_v, sem_u, sem_m):
    wid = lax.axis_index("s") * _NC + lax.axis_index("c")
    base = wid * _BPW

    pltpu.sync_copy(uidx_hbm.at[pl.ds(base, _BPW)], uidx_v)
    pltpu.sync_copy(midx_hbm.at[pl.ds(base, _BPW)], midx_v)
    cu = pltpu.async_copy(utab_hbm.at[uidx_v], urows_v, sem_u)
    cm = pltpu.async_copy(mtab_hbm.at[midx_v], mrows_v, sem_m)
    cu.wait()
    cm.wait()

    def chunk_body(c, carry):
        row_idx = c * _L + lax.iota(jnp.int32, _L)
        acc = jnp.zeros((_L,), jnp.float32)
        for d in range(EMBED_DIM):
            col = jnp.full((_L,), d, jnp.int32)
            u = plsc.load_gather(urows_v, [row_idx, col])
            m = plsc.load_gather(mrows_v, [row_idx, col])
            acc = acc + u * m
        out_v[pl.ds(c * _L, _L)] = acc
        return carry

    lax.fori_loop(0, _CHUNKS, chunk_body, 0)
    pltpu.sync_copy(out_v, out_hbm.at[pl.ds(base, _BPW)])


def kernel(user_indices, movie_indices, user_table, movie_table):
    uidx = user_indices.astype(jnp.int32)
    midx = movie_indices.astype(jnp.int32)
    mesh = plsc.VectorSubcoreMesh(core_axis_name="c", subcore_axis_name="s")
    run = functools.partial(
        pl.kernel,
        mesh=mesh,
        out_type=jax.ShapeDtypeStruct((BATCH,), jnp.float32),
        scratch_types=[
            pltpu.VMEM((_BPW,), jnp.int32),
            pltpu.VMEM((_BPW,), jnp.int32),
            pltpu.VMEM((_BPW, EMBED_DIM), jnp.float32),
            pltpu.VMEM((_BPW, EMBED_DIM), jnp.float32),
            pltpu.VMEM((_BPW,), jnp.float32),
            pltpu.SemaphoreType.DMA,
            pltpu.SemaphoreType.DMA,
        ],
        compiler_params=pltpu.CompilerParams(
            needs_layout_passes=False, use_tc_tiling_on_sc=False),
    )(_sc_body)
    return run(uidx, midx, user_table, movie_table)
